# Initial kernel scaffold; baseline (speedup 1.0000x reference)
import jax
import jax.numpy as jnp
from jax.experimental import pallas as pl


def kernel(nodes, edge_index, ff_w1, ff_b1, ff_g1, ff_be1, ff_w2, ff_b2, ff_g2, ff_be2, s1_ws, s1_bs, s1_wn, bn1_g, bn1_b, s2_ws, s2_bs, s2_wn, bn2_g, bn2_b):
    raise NotImplementedError("write your pallas kernel here")


# speedup vs baseline: 3.9575x; 3.9575x over previous
import jax
import jax.numpy as jnp
from jax import lax
from jax.experimental import pallas as pl
from jax.experimental.pallas import tpu as pltpu
from jax.experimental.pallas import tpu_sc as plsc

N = 10000
D = 128
NC = 2
NS = 16
NW = NC * NS
EB = 128
NP = 10112
ZR = 128


def _sc_pass(n_blocks, gather_h):
    mesh = plsc.VectorSubcoreMesh(
        core_axis_name="c", subcore_axis_name="s",
        num_cores=NC, num_subcores=NS)
    out_type = jax.ShapeDtypeStruct((NC * NP, D), jnp.float32)
    scratch = [
        pltpu.VMEM((EB,), jnp.int32),
        pltpu.VMEM((EB,), jnp.int32),
        pltpu.VMEM((EB, D), jnp.float32),
        pltpu.VMEM_SHARED((NP, D), jnp.float32),
        pltpu.SemaphoreType.DMA,
    ]

    def body(h_hbm, src_hbm, dst_hbm, zrow_hbm, one_hbm, acc_out,
             src_v, dst_v, rows_v, acc_sp, sem):
        cid = lax.axis_index("c")
        sid = lax.axis_index("s")
        wid = cid * NS + sid

        def fill_idx(base):
            for j in range(EB // 16):
                src_v[pl.ds(j * 16, 16)] = base + j * 16 + lax.iota(
                    jnp.int32, 16)

        pltpu.sync_copy(zrow_hbm, rows_v)
        for k in range(5):
            def zchunk(k=k):
                fill_idx(sid * 640 + k * ZR)
                pltpu.sync_copy(rows_v, acc_sp.at[src_v])
            if k < 4:
                zchunk()
            else:
                pl.when(sid < NS - 1)(zchunk)
        if not gather_h:
            pltpu.sync_copy(one_hbm, rows_v)
        plsc.subcore_barrier()

        def eblk(i, _):
            base = (wid * n_blocks + i) * EB
            pltpu.sync_copy(dst_hbm.at[pl.ds(base, EB)], dst_v)
            if gather_h:
                pltpu.sync_copy(src_hbm.at[pl.ds(base, EB)], src_v)
                pltpu.async_copy(h_hbm.at[src_v], rows_v, sem).wait()
            pltpu.sync_copy(rows_v, acc_sp.at[dst_v], add=True)
            return ()

        lax.fori_loop(0, n_blocks, eblk, ())
        plsc.subcore_barrier()

        for k in range(5):
            def wchunk(k=k):
                row0 = sid * 640 + k * ZR
                fill_idx(row0)
                pltpu.async_copy(acc_sp.at[src_v], rows_v, sem).wait()
                pltpu.sync_copy(
                    rows_v, acc_out.at[pl.ds(cid * NP + row0, ZR)])
            if k < 4:
                wchunk()
            else:
                pl.when(sid < NS - 1)(wchunk)

    return pl.kernel(body, out_type=out_type, mesh=mesh,
                     scratch_types=scratch)


def _bn_tc(x, g, b):
    m = jnp.mean(x, axis=0, keepdims=True)
    v = jnp.mean((x - m) * (x - m), axis=0, keepdims=True)
    return g * (x - m) * lax.rsqrt(v + 1e-5) + b


def _ffn_body(x_ref, w1_ref, b1_ref, g1_ref, be1_ref,
              w2_ref, b2_ref, g2_ref, be2_ref, o_ref):
    dn = (((1,), (1,)), ((), ()))
    x = x_ref[...]
    t = lax.dot_general(x, w1_ref[...], dn,
                        preferred_element_type=jnp.float32) + b1_ref[...]
    t = jnp.maximum(_bn_tc(t, g1_ref[...], be1_ref[...]), 0.0)
    t = lax.dot_general(t, w2_ref[...], dn,
                        preferred_element_type=jnp.float32) + b2_ref[...]
    o_ref[...] = jnp.maximum(_bn_tc(t, g2_ref[...], be2_ref[...]), 0.0)


def _sage_body(h_ref, agg_ref, deg_ref, ws_ref, bs_ref, wn_ref,
               g_ref, b_ref, o_ref):
    dn = (((1,), (1,)), ((), ()))
    agg = agg_ref[0:N, :] + agg_ref[NP:NP + N, :]
    deg = deg_ref[0:N, 0:1] + deg_ref[NP:NP + N, 0:1]
    hn = agg * (1.0 / jnp.maximum(deg, 1.0))
    z = (lax.dot_general(h_ref[...], ws_ref[...], dn,
                         preferred_element_type=jnp.float32)
         + bs_ref[...]
         + lax.dot_general(hn, wn_ref[...], dn,
                           preferred_element_type=jnp.float32))
    z = _bn_tc(z, g_ref[...], b_ref[...])
    o_ref[...] = jnp.where(z >= 0.0, z, 0.01 * z)


_ffn_call = pl.pallas_call(
    _ffn_body, out_shape=jax.ShapeDtypeStruct((N, D), jnp.float32))
_sage_call = pl.pallas_call(
    _sage_body, out_shape=jax.ShapeDtypeStruct((N, D), jnp.float32))


def kernel(nodes, edge_index, ff_w1, ff_b1, ff_g1, ff_be1, ff_w2, ff_b2,
           ff_g2, ff_be2, s1_ws, s1_bs, s1_wn, bn1_g, bn1_b, s2_ws, s2_bs,
           s2_wn, bn2_g, bn2_b):
    src = edge_index[0]
    dst = edge_index[1]
    e = src.shape[0]
    n_blocks = -(-e // (NW * EB))
    epad = NW * EB * n_blocks
    if epad != e:
        pad = epad - e
        src = jnp.concatenate([src, jnp.zeros((pad,), jnp.int32)])
        dst = jnp.concatenate([dst, jnp.full((pad,), N, jnp.int32)])
    zrow = jnp.zeros((ZR, D), jnp.float32)
    onerow = jnp.ones((ZR, D), jnp.float32)

    deg = _sc_pass(n_blocks, False)(nodes, src, dst, zrow, onerow)
    h = _ffn_call(nodes, ff_w1, ff_b1, ff_g1, ff_be1,
                  ff_w2, ff_b2, ff_g2, ff_be2)

    agg1 = _sc_pass(n_blocks, True)(h, src, dst, zrow, onerow)
    h1 = _sage_call(h, agg1, deg, s1_ws, s1_bs, s1_wn, bn1_g, bn1_b)

    agg2 = _sc_pass(n_blocks, True)(h1, src, dst, zrow, onerow)
    out = _sage_call(h1, agg2, deg, s2_ws, s2_bs, s2_wn, bn2_g, bn2_b)
    return out
